# baseline (speedup 1.0000x reference)
import functools

import jax
import jax.numpy as jnp
from jax.experimental import pallas as pl
from jax.experimental.pallas import tpu as pltpu


def _fused_kernel(x_ref, w_ref, b_ref, fcw_ref, fcb_ref, out_ref, *, W, TB):
    T = x_ref.shape[1]
    tmod = jax.lax.broadcasted_iota(jnp.int32, (T, 1), 0) % 3
    is1 = tmod == 1
    is2 = tmod == 2

    def sel(v):
        return jnp.where(is2, v[2], jnp.where(is1, v[1], v[0]))

    x = x_ref[...].astype(jnp.bfloat16)
    a = x * sel(w_ref[:, W - 1])[None].astype(jnp.bfloat16)
    for k in range(W - 1):
        d = W - 1 - k
        wk = sel(w_ref[:, k]).astype(jnp.bfloat16)
        contrib = x[:, : T - d, :] * wk[None, d:, :]
        a = a + jnp.pad(contrib, ((0, 0), (d, 0), (0, 0)))

    a = a + sel(b_ref[...])[None].astype(jnp.bfloat16)
    C = x.shape[2]
    y = jax.lax.dot_general(
        a.reshape(TB * T, C), fcw_ref[...].astype(jnp.bfloat16),
        (((1,), (1,)), ((), ())),
        preferred_element_type=jnp.float32)
    out_ref[...] = (y + fcb_ref[...]).reshape(TB, T, C).astype(out_ref.dtype)


def kernel(x, rtg_w, rtg_b, obs_w, obs_b, act_w, act_b, fc_w, fc_b):
    B, T, C = x.shape
    W = rtg_w.shape[1]

    batch_blocks = 8 if B % 8 == 0 else (2 if B % 2 == 0 else 1)
    TB = B // batch_blocks

    w_stack = jnp.transpose(jnp.stack([rtg_w, obs_w, act_w]), (0, 2, 1))
    b_stack = jnp.stack([rtg_b, obs_b, act_b])

    out = pl.pallas_call(
        functools.partial(_fused_kernel, W=W, TB=TB),
        out_shape=jax.ShapeDtypeStruct((B, T, C), x.dtype),
        grid=(batch_blocks,),
        in_specs=[
            pl.BlockSpec((TB, T, C), lambda i: (i, 0, 0)),
            pl.BlockSpec((3, W, C), lambda i: (0, 0, 0)),
            pl.BlockSpec((3, C), lambda i: (0, 0)),
            pl.BlockSpec((C, C), lambda i: (0, 0)),
            pl.BlockSpec((1, C), lambda i: (0, 0)),
        ],
        out_specs=pl.BlockSpec((TB, T, C), lambda i: (i, 0, 0)),
        compiler_params=pltpu.CompilerParams(
            dimension_semantics=("parallel",)),
    )(x, w_stack, b_stack, fc_w, fc_b.reshape(1, C))
    return out


# speedup vs baseline: 1.0667x; 1.0667x over previous
import functools

import jax
import jax.numpy as jnp
from jax.experimental import pallas as pl
from jax.experimental.pallas import tpu as pltpu


def _fused_kernel(x_ref, w_ref, b_ref, fcw_ref, fcb_ref, out_ref, *, W, TB):
    T = x_ref.shape[1]
    tmod = jax.lax.broadcasted_iota(jnp.int32, (T, 1), 0) % 3
    is1 = tmod == 1
    is2 = tmod == 2

    def sel(v):
        return jnp.where(is2, v[2], jnp.where(is1, v[1], v[0]))

    x = x_ref[...].astype(jnp.bfloat16)
    a = x * sel(w_ref[:, W - 1])[None].astype(jnp.bfloat16)
    for k in range(W - 1):
        d = W - 1 - k
        wk = sel(w_ref[:, k]).astype(jnp.bfloat16)
        contrib = x[:, : T - d, :] * wk[None, d:, :]
        a = a + jnp.pad(contrib, ((0, 0), (d, 0), (0, 0)))

    a = a + sel(b_ref[...])[None].astype(jnp.bfloat16)
    C = x.shape[2]
    y = jax.lax.dot_general(
        a.reshape(TB * T, C), fcw_ref[...].astype(jnp.bfloat16),
        (((1,), (1,)), ((), ())),
        preferred_element_type=jnp.float32)
    out_ref[...] = (y + fcb_ref[...]).reshape(TB, T, C).astype(out_ref.dtype)


def kernel(x, rtg_w, rtg_b, obs_w, obs_b, act_w, act_b, fc_w, fc_b):
    B, T, C = x.shape
    W = rtg_w.shape[1]

    batch_blocks = 4 if B % 4 == 0 else (2 if B % 2 == 0 else 1)
    TB = B // batch_blocks

    w_stack = jnp.transpose(jnp.stack([rtg_w, obs_w, act_w]), (0, 2, 1))
    b_stack = jnp.stack([rtg_b, obs_b, act_b])

    out = pl.pallas_call(
        functools.partial(_fused_kernel, W=W, TB=TB),
        out_shape=jax.ShapeDtypeStruct((B, T, C), x.dtype),
        grid=(batch_blocks,),
        in_specs=[
            pl.BlockSpec((TB, T, C), lambda i: (i, 0, 0)),
            pl.BlockSpec((3, W, C), lambda i: (0, 0, 0)),
            pl.BlockSpec((3, C), lambda i: (0, 0)),
            pl.BlockSpec((C, C), lambda i: (0, 0)),
            pl.BlockSpec((1, C), lambda i: (0, 0)),
        ],
        out_specs=pl.BlockSpec((TB, T, C), lambda i: (i, 0, 0)),
        compiler_params=pltpu.CompilerParams(
            dimension_semantics=("parallel",)),
    )(x, w_stack, b_stack, fc_w, fc_b.reshape(1, C))
    return out
